# baseline (speedup 1.0000x reference)
import jax
import jax.numpy as jnp
from jax import lax
from jax.experimental import pallas as pl
from jax.experimental.pallas import tpu as pltpu
from jax.experimental.pallas import tpu_sc as plsc

BATCH = 16384
EMB_DIM = 32
L = 16
NC, NS = 2, 16
NW = NC * NS
BPW = BATCH // NW
CHUNK = 128
NCHUNK = BPW // CHUNK


def _sc_body(uid_hbm, iid_hbm, uemb_hbm, iemb_hbm, bias_hbm, out_hbm,
             uid_v, iid_v, urows_v, irows_v, bias_v, out_v, sem):
    wid = lax.axis_index("s") * NC + lax.axis_index("c")
    base = wid * BPW
    crow = wid * NCHUNK

    pltpu.sync_copy(uid_hbm.at[pl.ds(crow, NCHUNK)], uid_v)
    pltpu.sync_copy(iid_hbm.at[pl.ds(crow, NCHUNK)], iid_v)
    pltpu.sync_copy(bias_hbm, bias_v)

    copies = []
    for j in range(NCHUNK):
        copies.append(pltpu.async_copy(
            uemb_hbm.at[uid_v.at[j]],
            urows_v.at[pl.ds(j * CHUNK, CHUNK)], sem))
        copies.append(pltpu.async_copy(
            iemb_hbm.at[iid_v.at[j]],
            irows_v.at[pl.ds(j * CHUNK, CHUNK)], sem))
    for c in copies:
        c.wait()

    iota = lax.iota(jnp.int32, L)
    bias16 = bias_v[...]

    def group(g, carry):
        row = iota + g * L
        acc = jnp.zeros((L,), jnp.float32)
        for d in range(EMB_DIM):
            col = jnp.full((L,), d, jnp.int32)
            u = plsc.load_gather(urows_v, [row, col])
            v = plsc.load_gather(irows_v, [row, col])
            acc = acc + u * v
        x = acc + bias16
        y = 1.0 / (1.0 + jnp.exp(-x))
        out_v[pl.ds(pl.multiple_of(g * L, L), L)] = y
        return carry

    lax.fori_loop(0, BPW // L, group, 0)
    pltpu.sync_copy(out_v, out_hbm.at[pl.ds(base, BPW)])


def kernel(user_id, item_id, user_emb, item_emb, bias):
    uid2 = user_id.reshape(NW * NCHUNK, CHUNK).astype(jnp.int32)
    iid2 = item_id.reshape(NW * NCHUNK, CHUNK).astype(jnp.int32)
    bias16 = jnp.full((L,), bias, jnp.float32)
    mesh = plsc.VectorSubcoreMesh(core_axis_name="c", subcore_axis_name="s")
    f = pl.kernel(
        _sc_body,
        mesh=mesh,
        compiler_params=pltpu.CompilerParams(
            needs_layout_passes=False, use_tc_tiling_on_sc=False),
        out_type=jax.ShapeDtypeStruct((BATCH,), jnp.float32),
        scratch_types=[
            pltpu.VMEM((NCHUNK, CHUNK), jnp.int32),
            pltpu.VMEM((NCHUNK, CHUNK), jnp.int32),
            pltpu.VMEM((BPW, EMB_DIM), jnp.float32),
            pltpu.VMEM((BPW, EMB_DIM), jnp.float32),
            pltpu.VMEM((L,), jnp.float32),
            pltpu.VMEM((BPW,), jnp.float32),
            pltpu.SemaphoreType.DMA,
        ],
    )
    out = f(uid2, iid2, user_emb, item_emb, bias16)
    return out.reshape(BATCH, 1)


# speedup vs baseline: 32.6428x; 32.6428x over previous
import jax
import jax.numpy as jnp
from jax import lax
from jax.experimental import pallas as pl
from jax.experimental.pallas import tpu as pltpu
from jax.experimental.pallas import tpu_sc as plsc

BATCH = 16384
EMB_DIM = 32
L = 16
NC, NS = 2, 16
NW = NC * NS
BPW = BATCH // NW
NR = BPW // L
H = 8


def _sc_body(uid_hbm, iid_hbm, utab_hbm, itab_hbm, bias_hbm, out_hbm,
             uid_v, iid_v, ubuf, ibuf, bias_v, out_v, sem):
    wid = lax.axis_index("s") * NC + lax.axis_index("c")
    base = wid * BPW

    pltpu.sync_copy(uid_hbm.at[pl.ds(wid * NR, NR)], uid_v)
    pltpu.sync_copy(iid_hbm.at[pl.ds(wid * NR, NR)], iid_v)
    pltpu.sync_copy(bias_hbm, bias_v)

    iota = lax.iota(jnp.int32, L)
    bias16 = bias_v[...]

    def row(r, carry):
        uvec = uid_v[r, :]
        ivec = iid_v[r, :]
        ulane = jnp.bitwise_and(uvec, 127)
        ilane = jnp.bitwise_and(ivec, 127)
        ucol = jnp.bitwise_and(uvec, -128)
        icol = jnp.bitwise_and(ivec, -128)
        res = jnp.zeros((L,), jnp.float32)
        for h in range(2):
            copies = []
            for l in range(H):
                cu = pl.multiple_of(ucol[h * H + l], 128)
                ci = pl.multiple_of(icol[h * H + l], 128)
                copies.append(pltpu.async_copy(
                    utab_hbm.at[:, pl.ds(cu, 128)], ubuf.at[l], sem))
                copies.append(pltpu.async_copy(
                    itab_hbm.at[:, pl.ds(ci, 128)], ibuf.at[l], sem))
            for c in copies:
                c.wait()
            for l in range(H):
                lu = jnp.full((L,), ulane[h * H + l], jnp.int32)
                li = jnp.full((L,), ilane[h * H + l], jnp.int32)
                lv = jnp.full((L,), l, jnp.int32)
                u0 = plsc.load_gather(ubuf, [lv, iota, lu])
                u1 = plsc.load_gather(ubuf, [lv, iota + L, lu])
                v0 = plsc.load_gather(ibuf, [lv, iota, li])
                v1 = plsc.load_gather(ibuf, [lv, iota + L, li])
                p = u0 * v0 + u1 * v1
                s = lax.reduce_sum_p.bind(p, axes=(0,))
                res = jnp.where(iota == (h * H + l), s, res)
        out_v[pl.ds(pl.multiple_of(r * L, L), L)] = res
        return carry

    lax.fori_loop(0, NR, row, 0)

    def sig(g, carry):
        x = out_v[pl.ds(pl.multiple_of(g * L, L), L)] + bias16
        out_v[pl.ds(pl.multiple_of(g * L, L), L)] = 1.0 / (1.0 + jnp.exp(-x))
        return carry

    lax.fori_loop(0, NR, sig, 0)
    pltpu.sync_copy(out_v, out_hbm.at[pl.ds(base, BPW)])


def kernel(user_id, item_id, user_emb, item_emb, bias):
    uid2 = user_id.reshape(NW * NR, L).astype(jnp.int32)
    iid2 = item_id.reshape(NW * NR, L).astype(jnp.int32)
    utab = user_emb.T
    itab = item_emb.T
    bias16 = jnp.full((L,), bias, jnp.float32)
    mesh = plsc.VectorSubcoreMesh(core_axis_name="c", subcore_axis_name="s")
    f = pl.kernel(
        _sc_body,
        mesh=mesh,
        compiler_params=pltpu.CompilerParams(
            needs_layout_passes=False, use_tc_tiling_on_sc=True),
        out_type=jax.ShapeDtypeStruct((BATCH,), jnp.float32),
        scratch_types=[
            pltpu.VMEM((NR, L), jnp.int32),
            pltpu.VMEM((NR, L), jnp.int32),
            pltpu.VMEM((H, EMB_DIM, 128), jnp.float32),
            pltpu.VMEM((H, EMB_DIM, 128), jnp.float32),
            pltpu.VMEM((L,), jnp.float32),
            pltpu.VMEM((BPW,), jnp.float32),
            pltpu.SemaphoreType.DMA,
        ],
    )
    out = f(uid2, iid2, utab, itab, bias16)
    return out.reshape(BATCH, 1)


# speedup vs baseline: 35.0862x; 1.0749x over previous
import jax
import jax.numpy as jnp
from jax import lax
from jax.experimental import pallas as pl
from jax.experimental.pallas import tpu as pltpu
from jax.experimental.pallas import tpu_sc as plsc

BATCH = 16384
EMB_DIM = 32
L = 16
NC, NS = 2, 16
NW = NC * NS
BPW = BATCH // NW
NR = BPW // L


def _sc_body(uid_hbm, iid_hbm, utab_hbm, itab_hbm, bias_hbm, out_hbm,
             uid_v, iid_v, buf, ustage, istage, bias_v, out_v, sems):
    wid = lax.axis_index("s") * NC + lax.axis_index("c")
    base = wid * BPW

    pltpu.sync_copy(uid_hbm.at[pl.ds(wid * NR, NR)], uid_v)
    pltpu.sync_copy(iid_hbm.at[pl.ds(wid * NR, NR)], iid_v)
    pltpu.sync_copy(bias_hbm, bias_v)

    iota = lax.iota(jnp.int32, L)
    bias16 = bias_v[...]

    def fetch(tab, col, l):
        pltpu.async_copy(
            tab.at[:, pl.ds(pl.multiple_of(col, 128), 128)],
            buf.at[l], sems.at[l])

    def drain(l):
        pltpu.make_async_copy(
            utab_hbm.at[:, pl.ds(0, 128)], buf.at[l], sems.at[l]).wait()

    def extract(lane, l, stage):
        lane16 = jnp.full((L,), lane, jnp.int32)
        l16 = jnp.full((L,), l, jnp.int32)
        e0 = plsc.load_gather(buf, [l16, iota, lane16])
        e1 = plsc.load_gather(buf, [l16, iota + L, lane16])
        stage[pl.ds(l * EMB_DIM, L)] = e0
        stage[pl.ds(l * EMB_DIM + L, L)] = e1

    uvec0 = uid_v[0, :]
    ucol0 = jnp.bitwise_and(uvec0, -128)
    for l in range(L):
        fetch(utab_hbm, ucol0[l], l)

    def row(r, carry):
        uvec = uid_v[r, :]
        ivec = iid_v[r, :]
        ulane = jnp.bitwise_and(uvec, 127)
        ilane = jnp.bitwise_and(ivec, 127)
        icol = jnp.bitwise_and(ivec, -128)
        for l in range(L):
            drain(l)
            extract(ulane[l], l, ustage)
            fetch(itab_hbm, icol[l], l)
        rn = jnp.where(r + 1 < NR, r + 1, r)
        uvecn = uid_v[rn, :]
        ucoln = jnp.bitwise_and(uvecn, -128)
        for l in range(L):
            drain(l)
            extract(ilane[l], l, istage)

            @pl.when(r + 1 < NR)
            def _():
                fetch(utab_hbm, ucoln[l], l)

        res = jnp.zeros((L,), jnp.float32)
        for l in range(L):
            u0 = ustage[pl.ds(l * EMB_DIM, L)]
            u1 = ustage[pl.ds(l * EMB_DIM + L, L)]
            v0 = istage[pl.ds(l * EMB_DIM, L)]
            v1 = istage[pl.ds(l * EMB_DIM + L, L)]
            p = u0 * v0 + u1 * v1
            s = lax.reduce_sum_p.bind(p, axes=(0,))
            res = jnp.where(iota == l, s, res)
        x = res + bias16
        y = 1.0 / (1.0 + jnp.exp(-x))
        out_v[pl.ds(pl.multiple_of(r * L, L), L)] = y
        return carry

    lax.fori_loop(0, NR, row, 0)
    pltpu.sync_copy(out_v, out_hbm.at[pl.ds(base, BPW)])


def kernel(user_id, item_id, user_emb, item_emb, bias):
    uid2 = user_id.reshape(NW * NR, L).astype(jnp.int32)
    iid2 = item_id.reshape(NW * NR, L).astype(jnp.int32)
    utab = user_emb.T
    itab = item_emb.T
    bias16 = jnp.full((L,), bias, jnp.float32)
    mesh = plsc.VectorSubcoreMesh(core_axis_name="c", subcore_axis_name="s")
    f = pl.kernel(
        _sc_body,
        mesh=mesh,
        compiler_params=pltpu.CompilerParams(
            needs_layout_passes=False, use_tc_tiling_on_sc=True),
        out_type=jax.ShapeDtypeStruct((BATCH,), jnp.float32),
        scratch_types=[
            pltpu.VMEM((NR, L), jnp.int32),
            pltpu.VMEM((NR, L), jnp.int32),
            pltpu.VMEM((L, EMB_DIM, 128), jnp.float32),
            pltpu.VMEM((L * EMB_DIM,), jnp.float32),
            pltpu.VMEM((L * EMB_DIM,), jnp.float32),
            pltpu.VMEM((L,), jnp.float32),
            pltpu.VMEM((BPW,), jnp.float32),
            pltpu.SemaphoreType.DMA((L,)),
        ],
    )
    out = f(uid2, iid2, utab, itab, bias16)
    return out.reshape(BATCH, 1)


# speedup vs baseline: 35.9747x; 1.0253x over previous
import jax
import jax.numpy as jnp
from jax import lax
from jax.experimental import pallas as pl
from jax.experimental.pallas import tpu as pltpu
from jax.experimental.pallas import tpu_sc as plsc

BATCH = 16384
EMB_DIM = 32
L = 16
NC, NS = 2, 16
NW = NC * NS
BPW = BATCH // NW
NR = BPW // L


def _sc_body(uid_hbm, iid_hbm, utab_hbm, itab_hbm, bias_hbm, out_hbm,
             uid_v, iid_v, buf, ustage, istage, bias_v, out_v, sems):
    wid = lax.axis_index("s") * NC + lax.axis_index("c")
    base = wid * BPW

    pltpu.sync_copy(uid_hbm.at[pl.ds(wid * NR, NR)], uid_v)
    pltpu.sync_copy(iid_hbm.at[pl.ds(wid * NR, NR)], iid_v)
    pltpu.sync_copy(bias_hbm, bias_v)

    iota = lax.iota(jnp.int32, L)
    bias16 = bias_v[...]

    def fetch(tab, col, l):
        c = pl.multiple_of(col, 128)
        for j in range(4):
            pltpu.async_copy(
                tab.at[pl.ds(j * 8, 8), pl.ds(c, 128)],
                buf.at[l, pl.ds(j * 8, 8)], sems.at[l])

    def drain(l):
        pltpu.make_async_copy(
            utab_hbm.at[:, pl.ds(0, 128)], buf.at[l], sems.at[l]).wait()

    def extract(lane, l, stage):
        lane16 = jnp.full((L,), lane, jnp.int32)
        l16 = jnp.full((L,), l, jnp.int32)
        e0 = plsc.load_gather(buf, [l16, iota, lane16])
        e1 = plsc.load_gather(buf, [l16, iota + L, lane16])
        stage[pl.ds(l * EMB_DIM, L)] = e0
        stage[pl.ds(l * EMB_DIM + L, L)] = e1

    uvec0 = uid_v[0, :]
    ucol0 = jnp.bitwise_and(uvec0, -128)
    for l in range(L):
        fetch(utab_hbm, ucol0[l], l)

    def row(r, carry):
        uvec = uid_v[r, :]
        ivec = iid_v[r, :]
        ulane = jnp.bitwise_and(uvec, 127)
        ilane = jnp.bitwise_and(ivec, 127)
        icol = jnp.bitwise_and(ivec, -128)
        for l in range(L):
            drain(l)
            extract(ulane[l], l, ustage)
            fetch(itab_hbm, icol[l], l)
        rn = jnp.where(r + 1 < NR, r + 1, r)
        uvecn = uid_v[rn, :]
        ucoln = jnp.bitwise_and(uvecn, -128)
        for l in range(L):
            drain(l)
            extract(ilane[l], l, istage)

            @pl.when(r + 1 < NR)
            def _():
                fetch(utab_hbm, ucoln[l], l)

        res = jnp.zeros((L,), jnp.float32)
        for l in range(L):
            u0 = ustage[pl.ds(l * EMB_DIM, L)]
            u1 = ustage[pl.ds(l * EMB_DIM + L, L)]
            v0 = istage[pl.ds(l * EMB_DIM, L)]
            v1 = istage[pl.ds(l * EMB_DIM + L, L)]
            p = u0 * v0 + u1 * v1
            s = lax.reduce_sum_p.bind(p, axes=(0,))
            res = jnp.where(iota == l, s, res)
        x = res + bias16
        y = 1.0 / (1.0 + jnp.exp(-x))
        out_v[pl.ds(pl.multiple_of(r * L, L), L)] = y
        return carry

    lax.fori_loop(0, NR, row, 0)
    pltpu.sync_copy(out_v, out_hbm.at[pl.ds(base, BPW)])


def kernel(user_id, item_id, user_emb, item_emb, bias):
    uid2 = user_id.reshape(NW * NR, L).astype(jnp.int32)
    iid2 = item_id.reshape(NW * NR, L).astype(jnp.int32)
    utab = user_emb.T
    itab = item_emb.T
    bias16 = jnp.full((L,), bias, jnp.float32)
    mesh = plsc.VectorSubcoreMesh(core_axis_name="c", subcore_axis_name="s")
    f = pl.kernel(
        _sc_body,
        mesh=mesh,
        compiler_params=pltpu.CompilerParams(
            needs_layout_passes=False, use_tc_tiling_on_sc=True),
        out_type=jax.ShapeDtypeStruct((BATCH,), jnp.float32),
        scratch_types=[
            pltpu.VMEM((NR, L), jnp.int32),
            pltpu.VMEM((NR, L), jnp.int32),
            pltpu.VMEM((L, EMB_DIM, 128), jnp.float32),
            pltpu.VMEM((L * EMB_DIM,), jnp.float32),
            pltpu.VMEM((L * EMB_DIM,), jnp.float32),
            pltpu.VMEM((L,), jnp.float32),
            pltpu.VMEM((BPW,), jnp.float32),
            pltpu.SemaphoreType.DMA((L,)),
        ],
    )
    out = f(uid2, iid2, utab, itab, bias16)
    return out.reshape(BATCH, 1)
